# baseline (speedup 1.0000x reference)
import jax
import jax.numpy as jnp
from jax import lax
from jax.experimental import pallas as pl
from jax.experimental.pallas import tpu as pltpu

N_CONCEPTS = 100000
D = 512
HID = 1024
K = 32
BLOCK = 10000
NBLK = N_CONCEPTS // BLOCK

_NEG_INF = float("-inf")
_BIG_I32 = 0x7FFFFFFF


def _gelu(x):
    return 0.5 * x * (1.0 + lax.erf(x * (2.0 ** -0.5)))


def _layernorm(x, g, b, eps=1e-5):
    m = jnp.mean(x, axis=-1, keepdims=True)
    v = jnp.mean((x - m) ** 2, axis=-1, keepdims=True)
    return (x - m) / jnp.sqrt(v + eps) * g + b


def _bdot(a, b):
    dn = (((1,), (1,)), ((), ()))
    return lax.dot_general(a.astype(jnp.bfloat16), b.astype(jnp.bfloat16),
                           dn, preferred_element_type=jnp.float32)


def _mlp_kernel(sv_ref, sd_ref, w1_ref, b1_ref, g1_ref, be1_ref, w2_ref,
                b2_ref, w3_ref, b3_ref, wg_ref, bg_ref, zn_ref):
    sv = sv_ref[...]
    h = _gelu(_bdot(sv, w1_ref[...]) + b1_ref[...])
    h = _layernorm(h, g1_ref[...], be1_ref[...])
    h = _gelu(_bdot(h, w2_ref[...]) + b2_ref[...])
    z = _bdot(h, w3_ref[...]) + b3_ref[...]
    gate_in = jnp.concatenate([sd_ref[...], z], axis=1)
    gate = jax.nn.sigmoid(_bdot(gate_in, wg_ref[...]) + bg_ref[...])
    z = z * gate
    nrm = jnp.sqrt(jnp.sum(z * z, axis=1, keepdims=True))
    zn_ref[...] = z / jnp.maximum(nrm, 1e-12)


def _sims_topk_kernel(c_ref, zn_ref, vals_ref, idx_ref, s_ref,
                      m1_ref, m2_ref, m3_ref, a1_ref, a2_ref, a3_ref):
    i = pl.program_id(0)

    @pl.when(i == 0)
    def _init():
        neg = jnp.full((1, BLOCK), _NEG_INF, jnp.float32)
        zero = jnp.zeros((1, BLOCK), jnp.int32)
        m1_ref[...] = neg
        m2_ref[...] = neg
        m3_ref[...] = neg
        a1_ref[...] = zero
        a2_ref[...] = zero
        a3_ref[...] = zero

    c = c_ref[...]
    n = jnp.sum(c * c, axis=1, keepdims=True)
    cn = c / jnp.maximum(jnp.sqrt(n), 1e-12)
    sims = _bdot(zn_ref[...], cn)
    s_ref[i, :] = sims[0, :]

    m1, m2, m3 = m1_ref[...], m2_ref[...], m3_ref[...]
    a1, a2, a3 = a1_ref[...], a2_ref[...], a3_ref[...]
    gt1 = sims > m1
    gt2 = sims > m2
    gt3 = sims > m3
    m1_ref[...] = jnp.where(gt1, sims, m1)
    m2_ref[...] = jnp.where(gt1, m1, jnp.where(gt2, sims, m2))
    m3_ref[...] = jnp.where(gt1 | gt2, m2, jnp.where(gt3, sims, m3))
    a1_ref[...] = jnp.where(gt1, i, a1)
    a2_ref[...] = jnp.where(gt1, a1, jnp.where(gt2, i, a2))
    a3_ref[...] = jnp.where(gt1 | gt2, a2, jnp.where(gt3, i, a3))

    @pl.when(i == NBLK - 1)
    def _select():
        cio = lax.broadcasted_iota(jnp.int32, (1, BLOCK), 1)
        kio = lax.broadcasted_iota(jnp.int32, (1, K), 1)

        def fast_body(k, carry):
            vals, idxs, m1, m2, m3, a1, a2, a3, cnt = carry
            m = jnp.max(m1)
            gidx = a1 * BLOCK + cio
            sel = jnp.min(jnp.where(m1 == m, gidx, _BIG_I32))
            hit = gidx == sel
            vals = jnp.where(kio == k, m, vals)
            idxs = jnp.where(kio == k, sel, idxs)
            m1 = jnp.where(hit, m2, m1)
            a1 = jnp.where(hit, a2, a1)
            m2 = jnp.where(hit, m3, m2)
            a2 = jnp.where(hit, a3, a2)
            m3 = jnp.where(hit, _NEG_INF, m3)
            cnt = cnt + jnp.where(hit, 1, 0)
            return vals, idxs, m1, m2, m3, a1, a2, a3, cnt

        vals0 = jnp.full((1, K), _NEG_INF, jnp.float32)
        idxs0 = jnp.zeros((1, K), jnp.int32)
        cnt0 = jnp.zeros((1, BLOCK), jnp.int32)
        out = lax.fori_loop(0, K, fast_body,
                            (vals0, idxs0, m1_ref[...], m2_ref[...],
                             m3_ref[...], a1_ref[...], a2_ref[...],
                             a3_ref[...], cnt0))
        fvals, fidxs, cnt = out[0], out[1], out[8]
        suspect = jnp.max(cnt) >= 3

        def slow_path(_):
            row = lax.broadcasted_iota(jnp.int32, (NBLK, BLOCK), 0)
            col = lax.broadcasted_iota(jnp.int32, (NBLK, BLOCK), 1)
            lin = row * BLOCK + col

            def body(k, carry):
                vals, idxs = carry
                s = s_ref[...]
                m = jnp.max(s)
                idx = jnp.min(jnp.where(s == m, lin, _BIG_I32))
                s_ref[...] = jnp.where(lin == idx, _NEG_INF, s)
                vals = jnp.where(kio == k, m, vals)
                idxs = jnp.where(kio == k, idx, idxs)
                return vals, idxs

            return lax.fori_loop(0, K, body, (vals0, idxs0))

        vals, idxs = lax.cond(suspect, slow_path,
                              lambda _: (fvals, fidxs), None)
        vals_ref[...] = vals
        idx_ref[...] = idxs


@jax.jit
def _run(slot_vec, state_delta, W1, b1, g1, be1, W2, b2, W3, b3, Wg, bg,
         concept_embs):
    sv = slot_vec.reshape(1, D)
    sd = state_delta.reshape(1, D)
    zn = pl.pallas_call(
        _mlp_kernel,
        out_shape=jax.ShapeDtypeStruct((1, D), jnp.float32),
    )(sv, sd, W1, b1.reshape(1, HID), g1.reshape(1, HID),
      be1.reshape(1, HID), W2, b2.reshape(1, HID), W3, b3.reshape(1, D),
      Wg, bg.reshape(1, D))

    vals, idxs = pl.pallas_call(
        _sims_topk_kernel,
        grid=(NBLK,),
        in_specs=[
            pl.BlockSpec((BLOCK, D), lambda i: (i, 0)),
            pl.BlockSpec((1, D), lambda i: (0, 0)),
        ],
        out_specs=[
            pl.BlockSpec((1, K), lambda i: (0, 0)),
            pl.BlockSpec((1, K), lambda i: (0, 0)),
        ],
        out_shape=[
            jax.ShapeDtypeStruct((1, K), jnp.float32),
            jax.ShapeDtypeStruct((1, K), jnp.int32),
        ],
        scratch_shapes=[
            pltpu.VMEM((NBLK, BLOCK), jnp.float32),
            pltpu.VMEM((1, BLOCK), jnp.float32),
            pltpu.VMEM((1, BLOCK), jnp.float32),
            pltpu.VMEM((1, BLOCK), jnp.float32),
            pltpu.VMEM((1, BLOCK), jnp.int32),
            pltpu.VMEM((1, BLOCK), jnp.int32),
            pltpu.VMEM((1, BLOCK), jnp.int32),
        ],
    )(concept_embs, zn)
    return vals.reshape(K), idxs.reshape(K)


def kernel(slot_vec, state_delta, W1, b1, g1, be1, W2, b2, W3, b3, Wg, bg,
           concept_embs, top_k):
    return _run(slot_vec, state_delta, W1, b1, g1, be1, W2, b2, W3, b3,
                Wg, bg, concept_embs)


# speedup vs baseline: 1.0309x; 1.0309x over previous
import jax
import jax.numpy as jnp
from jax import lax
from jax.experimental import pallas as pl
from jax.experimental.pallas import tpu as pltpu

N_CONCEPTS = 100000
D = 512
HID = 1024
K = 32
BLOCK = 5000
NBLK = N_CONCEPTS // BLOCK

_NEG_INF = float("-inf")
_BIG_I32 = 0x7FFFFFFF


def _gelu(x):
    return 0.5 * x * (1.0 + lax.erf(x * (2.0 ** -0.5)))


def _layernorm(x, g, b, eps=1e-5):
    m = jnp.mean(x, axis=-1, keepdims=True)
    v = jnp.mean((x - m) ** 2, axis=-1, keepdims=True)
    return (x - m) / jnp.sqrt(v + eps) * g + b


def _bdot(a, b):
    dn = (((1,), (1,)), ((), ()))
    return lax.dot_general(a.astype(jnp.bfloat16), b.astype(jnp.bfloat16),
                           dn, preferred_element_type=jnp.float32)


def _mlp_kernel(sv_ref, sd_ref, w1_ref, b1_ref, g1_ref, be1_ref, w2_ref,
                b2_ref, w3_ref, b3_ref, wg_ref, bg_ref, zn_ref):
    sv = sv_ref[...]
    h = _gelu(_bdot(sv, w1_ref[...]) + b1_ref[...])
    h = _layernorm(h, g1_ref[...], be1_ref[...])
    h = _gelu(_bdot(h, w2_ref[...]) + b2_ref[...])
    z = _bdot(h, w3_ref[...]) + b3_ref[...]
    gate_in = jnp.concatenate([sd_ref[...], z], axis=1)
    gate = jax.nn.sigmoid(_bdot(gate_in, wg_ref[...]) + bg_ref[...])
    z = z * gate
    nrm = jnp.sqrt(jnp.sum(z * z, axis=1, keepdims=True))
    zn_ref[...] = z / jnp.maximum(nrm, 1e-12)


def _sims_topk_kernel(c_ref, sv_ref, sd_ref, w1_ref, b1_ref, g1_ref,
                      be1_ref, w2_ref, b2_ref, w3_ref, b3_ref, wg_ref,
                      bg_ref, vals_ref, idx_ref, s_ref, zn_ref,
                      m1_ref, m2_ref, m3_ref, a1_ref, a2_ref, a3_ref):
    i = pl.program_id(0)

    @pl.when(i == 0)
    def _init():
        neg = jnp.full((1, BLOCK), _NEG_INF, jnp.float32)
        zero = jnp.zeros((1, BLOCK), jnp.int32)
        m1_ref[...] = neg
        m2_ref[...] = neg
        m3_ref[...] = neg
        a1_ref[...] = zero
        a2_ref[...] = zero
        a3_ref[...] = zero
        _mlp_kernel(sv_ref, sd_ref, w1_ref, b1_ref, g1_ref, be1_ref,
                    w2_ref, b2_ref, w3_ref, b3_ref, wg_ref, bg_ref, zn_ref)

    c = c_ref[...]
    n = jnp.sum(c * c, axis=1, keepdims=True)
    cn = c / jnp.maximum(jnp.sqrt(n), 1e-12)
    sims = _bdot(zn_ref[...], cn)
    s_ref[i, :] = sims[0, :]

    m1, m2, m3 = m1_ref[...], m2_ref[...], m3_ref[...]
    a1, a2, a3 = a1_ref[...], a2_ref[...], a3_ref[...]
    gt1 = sims > m1
    gt2 = sims > m2
    gt3 = sims > m3
    m1_ref[...] = jnp.where(gt1, sims, m1)
    m2_ref[...] = jnp.where(gt1, m1, jnp.where(gt2, sims, m2))
    m3_ref[...] = jnp.where(gt1 | gt2, m2, jnp.where(gt3, sims, m3))
    a1_ref[...] = jnp.where(gt1, i, a1)
    a2_ref[...] = jnp.where(gt1, a1, jnp.where(gt2, i, a2))
    a3_ref[...] = jnp.where(gt1 | gt2, a2, jnp.where(gt3, i, a3))

    @pl.when(i == NBLK - 1)
    def _select():
        cio = lax.broadcasted_iota(jnp.int32, (1, BLOCK), 1)
        kio = lax.broadcasted_iota(jnp.int32, (1, K), 1)

        def fast_body(k, carry):
            vals, idxs, m1, m2, m3, a1, a2, a3, cnt = carry
            m = jnp.max(m1)
            gidx = a1 * BLOCK + cio
            sel = jnp.min(jnp.where(m1 == m, gidx, _BIG_I32))
            hit = gidx == sel
            vals = jnp.where(kio == k, m, vals)
            idxs = jnp.where(kio == k, sel, idxs)
            m1 = jnp.where(hit, m2, m1)
            a1 = jnp.where(hit, a2, a1)
            m2 = jnp.where(hit, m3, m2)
            a2 = jnp.where(hit, a3, a2)
            m3 = jnp.where(hit, _NEG_INF, m3)
            cnt = cnt + jnp.where(hit, 1, 0)
            return vals, idxs, m1, m2, m3, a1, a2, a3, cnt

        vals0 = jnp.full((1, K), _NEG_INF, jnp.float32)
        idxs0 = jnp.zeros((1, K), jnp.int32)
        cnt0 = jnp.zeros((1, BLOCK), jnp.int32)
        out = lax.fori_loop(0, K, fast_body,
                            (vals0, idxs0, m1_ref[...], m2_ref[...],
                             m3_ref[...], a1_ref[...], a2_ref[...],
                             a3_ref[...], cnt0))
        fvals, fidxs, cnt = out[0], out[1], out[8]
        suspect = jnp.max(cnt) >= 3

        def slow_path(_):
            row = lax.broadcasted_iota(jnp.int32, (NBLK, BLOCK), 0)
            col = lax.broadcasted_iota(jnp.int32, (NBLK, BLOCK), 1)
            lin = row * BLOCK + col

            def body(k, carry):
                vals, idxs = carry
                s = s_ref[...]
                m = jnp.max(s)
                idx = jnp.min(jnp.where(s == m, lin, _BIG_I32))
                s_ref[...] = jnp.where(lin == idx, _NEG_INF, s)
                vals = jnp.where(kio == k, m, vals)
                idxs = jnp.where(kio == k, idx, idxs)
                return vals, idxs

            return lax.fori_loop(0, K, body, (vals0, idxs0))

        vals, idxs = lax.cond(suspect, slow_path,
                              lambda _: (fvals, fidxs), None)
        vals_ref[...] = vals
        idx_ref[...] = idxs


@jax.jit
def _run(slot_vec, state_delta, W1, b1, g1, be1, W2, b2, W3, b3, Wg, bg,
         concept_embs):
    sv = slot_vec.reshape(1, D)
    sd = state_delta.reshape(1, D)
    full = lambda shape: pl.BlockSpec(shape, lambda i: tuple(0 for _ in shape))

    vals, idxs = pl.pallas_call(
        _sims_topk_kernel,
        grid=(NBLK,),
        in_specs=[
            pl.BlockSpec((BLOCK, D), lambda i: (i, 0)),
            full((1, D)), full((1, D)),
            full((HID, D)), full((1, HID)), full((1, HID)), full((1, HID)),
            full((HID, HID)), full((1, HID)),
            full((D, HID)), full((1, D)),
            full((D, HID)), full((1, D)),
        ],
        out_specs=[
            pl.BlockSpec((1, K), lambda i: (0, 0)),
            pl.BlockSpec((1, K), lambda i: (0, 0)),
        ],
        out_shape=[
            jax.ShapeDtypeStruct((1, K), jnp.float32),
            jax.ShapeDtypeStruct((1, K), jnp.int32),
        ],
        scratch_shapes=[
            pltpu.VMEM((NBLK, BLOCK), jnp.float32),
            pltpu.VMEM((1, D), jnp.float32),
            pltpu.VMEM((1, BLOCK), jnp.float32),
            pltpu.VMEM((1, BLOCK), jnp.float32),
            pltpu.VMEM((1, BLOCK), jnp.float32),
            pltpu.VMEM((1, BLOCK), jnp.int32),
            pltpu.VMEM((1, BLOCK), jnp.int32),
            pltpu.VMEM((1, BLOCK), jnp.int32),
        ],
    )(concept_embs, sv, sd, W1, b1.reshape(1, HID), g1.reshape(1, HID),
      be1.reshape(1, HID), W2, b2.reshape(1, HID), W3, b3.reshape(1, D),
      Wg, bg.reshape(1, D))
    return vals.reshape(K), idxs.reshape(K)


def kernel(slot_vec, state_delta, W1, b1, g1, be1, W2, b2, W3, b3, Wg, bg,
           concept_embs, top_k):
    return _run(slot_vec, state_delta, W1, b1, g1, be1, W2, b2, W3, b3,
                Wg, bg, concept_embs)


# speedup vs baseline: 1.0582x; 1.0265x over previous
import jax
import jax.numpy as jnp
from jax import lax
from jax.experimental import pallas as pl
from jax.experimental.pallas import tpu as pltpu

N_CONCEPTS = 100000
D = 512
HID = 1024
K = 32
BLOCK = 5000
NBLK = N_CONCEPTS // BLOCK

_NEG_INF = float("-inf")
_BIG_I32 = 0x7FFFFFFF


def _gelu(x):
    return 0.5 * x * (1.0 + lax.erf(x * (2.0 ** -0.5)))


def _layernorm(x, g, b, eps=1e-5):
    m = jnp.mean(x, axis=-1, keepdims=True)
    v = jnp.mean((x - m) ** 2, axis=-1, keepdims=True)
    return (x - m) / jnp.sqrt(v + eps) * g + b


def _bdot(a, b):
    dn = (((1,), (1,)), ((), ()))
    return lax.dot_general(a.astype(jnp.bfloat16), b.astype(jnp.bfloat16),
                           dn, preferred_element_type=jnp.float32)


def _mlp_kernel(sv_ref, sd_ref, w1_ref, b1_ref, g1_ref, be1_ref, w2_ref,
                b2_ref, w3_ref, b3_ref, wg_ref, bg_ref, zn_ref):
    sv = sv_ref[...]
    h = _gelu(_bdot(sv, w1_ref[...]) + b1_ref[...])
    h = _layernorm(h, g1_ref[...], be1_ref[...])
    h = _gelu(_bdot(h, w2_ref[...]) + b2_ref[...])
    z = _bdot(h, w3_ref[...]) + b3_ref[...]
    gate_in = jnp.concatenate([sd_ref[...], z], axis=1)
    gate = jax.nn.sigmoid(_bdot(gate_in, wg_ref[...]) + bg_ref[...])
    z = z * gate
    nrm = jnp.sqrt(jnp.sum(z * z, axis=1, keepdims=True))
    zn_ref[...] = z / jnp.maximum(nrm, 1e-12)


NH = NBLK // 2


def _sims_topk_kernel(c0_ref, c1_ref, sv_ref, sd_ref, w1_ref, b1_ref, g1_ref,
                      be1_ref, w2_ref, b2_ref, w3_ref, b3_ref, wg_ref,
                      bg_ref, vals_ref, idx_ref, s_ref, zn_ref,
                      m1_ref, m2_ref, m3_ref, a1_ref, a2_ref, a3_ref):
    i = pl.program_id(0)

    @pl.when(i == 0)
    def _init():
        neg = jnp.full((1, BLOCK), _NEG_INF, jnp.float32)
        zero = jnp.zeros((1, BLOCK), jnp.int32)
        m1_ref[...] = neg
        m2_ref[...] = neg
        m3_ref[...] = neg
        a1_ref[...] = zero
        a2_ref[...] = zero
        a3_ref[...] = zero
        _mlp_kernel(sv_ref, sd_ref, w1_ref, b1_ref, g1_ref, be1_ref,
                    w2_ref, b2_ref, w3_ref, b3_ref, wg_ref, bg_ref, zn_ref)

    for c_ref, row in ((c0_ref, i), (c1_ref, i + NH)):
        c = c_ref[...]
        n = jnp.sum(c * c, axis=1, keepdims=True)
        cn = c / jnp.maximum(jnp.sqrt(n), 1e-12)
        sims = _bdot(zn_ref[...], cn)
        s_ref[row, :] = sims[0, :]

        m1, m2, m3 = m1_ref[...], m2_ref[...], m3_ref[...]
        a1, a2, a3 = a1_ref[...], a2_ref[...], a3_ref[...]
        gt1 = sims > m1
        gt2 = sims > m2
        gt3 = sims > m3
        m1_ref[...] = jnp.where(gt1, sims, m1)
        m2_ref[...] = jnp.where(gt1, m1, jnp.where(gt2, sims, m2))
        m3_ref[...] = jnp.where(gt1 | gt2, m2, jnp.where(gt3, sims, m3))
        a1_ref[...] = jnp.where(gt1, row, a1)
        a2_ref[...] = jnp.where(gt1, a1, jnp.where(gt2, row, a2))
        a3_ref[...] = jnp.where(gt1 | gt2, a2, jnp.where(gt3, row, a3))

    @pl.when(i == NH - 1)
    def _select():
        cio = lax.broadcasted_iota(jnp.int32, (1, BLOCK), 1)
        kio = lax.broadcasted_iota(jnp.int32, (1, K), 1)

        def fast_body(k, carry):
            vals, idxs, m1, m2, m3, a1, a2, a3, cnt = carry
            m = jnp.max(m1)
            gidx = a1 * BLOCK + cio
            sel = jnp.min(jnp.where(m1 == m, gidx, _BIG_I32))
            hit = gidx == sel
            vals = jnp.where(kio == k, m, vals)
            idxs = jnp.where(kio == k, sel, idxs)
            m1 = jnp.where(hit, m2, m1)
            a1 = jnp.where(hit, a2, a1)
            m2 = jnp.where(hit, m3, m2)
            a2 = jnp.where(hit, a3, a2)
            m3 = jnp.where(hit, _NEG_INF, m3)
            cnt = cnt + jnp.where(hit, 1, 0)
            return vals, idxs, m1, m2, m3, a1, a2, a3, cnt

        vals0 = jnp.full((1, K), _NEG_INF, jnp.float32)
        idxs0 = jnp.zeros((1, K), jnp.int32)
        cnt0 = jnp.zeros((1, BLOCK), jnp.int32)
        out = lax.fori_loop(0, K, fast_body,
                            (vals0, idxs0, m1_ref[...], m2_ref[...],
                             m3_ref[...], a1_ref[...], a2_ref[...],
                             a3_ref[...], cnt0))
        fvals, fidxs, cnt = out[0], out[1], out[8]
        suspect = jnp.max(cnt) >= 3

        def slow_path(_):
            row = lax.broadcasted_iota(jnp.int32, (NBLK, BLOCK), 0)
            col = lax.broadcasted_iota(jnp.int32, (NBLK, BLOCK), 1)
            lin = row * BLOCK + col

            def body(k, carry):
                vals, idxs = carry
                s = s_ref[...]
                m = jnp.max(s)
                idx = jnp.min(jnp.where(s == m, lin, _BIG_I32))
                s_ref[...] = jnp.where(lin == idx, _NEG_INF, s)
                vals = jnp.where(kio == k, m, vals)
                idxs = jnp.where(kio == k, idx, idxs)
                return vals, idxs

            return lax.fori_loop(0, K, body, (vals0, idxs0))

        vals, idxs = lax.cond(suspect, slow_path,
                              lambda _: (fvals, fidxs), None)
        vals_ref[...] = vals
        idx_ref[...] = idxs


@jax.jit
def _run(slot_vec, state_delta, W1, b1, g1, be1, W2, b2, W3, b3, Wg, bg,
         concept_embs):
    sv = slot_vec.reshape(1, D)
    sd = state_delta.reshape(1, D)
    full = lambda shape: pl.BlockSpec(shape, lambda i: tuple(0 for _ in shape))

    vals, idxs = pl.pallas_call(
        _sims_topk_kernel,
        grid=(NBLK // 2,),
        in_specs=[
            pl.BlockSpec((BLOCK, D), lambda i: (i, 0)),
            pl.BlockSpec((BLOCK, D), lambda i: (i + NBLK // 2, 0)),
            full((1, D)), full((1, D)),
            full((HID, D)), full((1, HID)), full((1, HID)), full((1, HID)),
            full((HID, HID)), full((1, HID)),
            full((D, HID)), full((1, D)),
            full((D, HID)), full((1, D)),
        ],
        out_specs=[
            pl.BlockSpec((1, K), lambda i: (0, 0)),
            pl.BlockSpec((1, K), lambda i: (0, 0)),
        ],
        out_shape=[
            jax.ShapeDtypeStruct((1, K), jnp.float32),
            jax.ShapeDtypeStruct((1, K), jnp.int32),
        ],
        scratch_shapes=[
            pltpu.VMEM((NBLK, BLOCK), jnp.float32),
            pltpu.VMEM((1, D), jnp.float32),
            pltpu.VMEM((1, BLOCK), jnp.float32),
            pltpu.VMEM((1, BLOCK), jnp.float32),
            pltpu.VMEM((1, BLOCK), jnp.float32),
            pltpu.VMEM((1, BLOCK), jnp.int32),
            pltpu.VMEM((1, BLOCK), jnp.int32),
            pltpu.VMEM((1, BLOCK), jnp.int32),
        ],
    )(concept_embs, concept_embs, sv, sd, W1, b1.reshape(1, HID), g1.reshape(1, HID),
      be1.reshape(1, HID), W2, b2.reshape(1, HID), W3, b3.reshape(1, D),
      Wg, bg.reshape(1, D))
    return vals.reshape(K), idxs.reshape(K)


def kernel(slot_vec, state_delta, W1, b1, g1, be1, W2, b2, W3, b3, Wg, bg,
           concept_embs, top_k):
    return _run(slot_vec, state_delta, W1, b1, g1, be1, W2, b2, W3, b3,
                Wg, bg, concept_embs)
